# baseline (speedup 1.0000x reference)
import functools

import jax
import jax.numpy as jnp
from jax import lax
from jax.experimental import pallas as pl
from jax.experimental.pallas import tpu as pltpu
from jax.experimental.pallas import tpu_sc as plsc

_F32 = jnp.float32
_CH = 128
_NB = 4
_NC, _NS = 2, 16
_BLK = 2048


def _ceil_to(v, m):
    return -(-v // m) * m


def _rsqrt16(z):
    i = plsc.bitcast(z, jnp.int32)
    y = plsc.bitcast(jnp.int32(0x5F3759DF) - (i >> 1), _F32)
    for _ in range(3):
        y = y * (1.5 - 0.5 * z * y * y)
    return y


_MESH = plsc.VectorSubcoreMesh(core_axis_name="c", subcore_axis_name="s")
_SC_PARAMS = pltpu.CompilerParams(use_tc_tiling_on_sc=False,
                                  needs_layout_passes=False)


def _drain_slot(zero, rows, ssem, p):
    for j in range(_NB):
        pltpu.make_async_copy(zero.at[pl.ds(0, _CH)], rows.at[p, j],
                              ssem).wait()


def _edge_loop(uu, src2, dst2, zero, sidx, didx, rows, acc, gsem, ssem,
               first, nb, gather):
    def it(b, carry):
        p = jnp.remainder(b, 2)

        @pl.when(b >= 2)
        def _():
            _drain_slot(zero, rows, ssem, p)

        row0 = first + b * _NB
        pltpu.sync_copy(dst2.at[pl.ds(row0, _NB)], didx.at[p])
        if gather:
            pltpu.sync_copy(src2.at[pl.ds(row0, _NB)], sidx.at[p])
            ds = [pltpu.async_copy(uu.at[sidx.at[p, j]], rows.at[p, j], gsem)
                  for j in range(_NB)]
            for d in ds:
                d.wait()
        for j in range(_NB):
            pltpu.async_copy(rows.at[p, j], acc.at[didx.at[p, j]],
                             ssem, add=True)
        return carry

    lax.fori_loop(0, nb, it, 0)
    for p in range(2):
        _drain_slot(zero, rows, ssem, p)


@functools.lru_cache(maxsize=None)
def _sc_deg(n_pad, e_pad):
    rows_pt = n_pad // _NS
    nch = e_pad // (_NC * _NS * _CH)

    def body(ones, src2, dst2, zero, out, sidx, didx, rows, acc, gsem, ssem):
        c = lax.axis_index("c")
        s = lax.axis_index("s")
        pltpu.sync_copy(zero, acc.at[pl.ds(s * rows_pt, rows_pt)])
        for p in range(2):
            for j in range(_NB):
                pltpu.sync_copy(ones, rows.at[p, j])
        plsc.subcore_barrier()
        _edge_loop(None, src2, dst2, zero, sidx, didx, rows, acc, gsem, ssem,
                   (c * _NS + s) * nch, nch // _NB, False)
        plsc.subcore_barrier()
        off = s * rows_pt
        pltpu.sync_copy(acc.at[pl.ds(off, rows_pt)],
                        out.at[c, pl.ds(off, rows_pt)])

    return pl.kernel(
        body,
        out_type=jax.ShapeDtypeStruct((_NC, n_pad), _F32),
        mesh=_MESH,
        compiler_params=_SC_PARAMS,
        scratch_types=[
            pltpu.VMEM((2, _NB, _CH), jnp.int32),
            pltpu.VMEM((2, _NB, _CH), jnp.int32),
            pltpu.VMEM((2, _NB, _CH), _F32),
            pltpu.VMEM_SHARED((n_pad,), _F32),
            pltpu.SemaphoreType.DMA,
            pltpu.SemaphoreType.DMA,
        ],
    )


@functools.lru_cache(maxsize=None)
def _sc_pass0(n_pad, e_pad):
    rows_pt = n_pad // _NS
    nch = e_pad // (_NC * _NS * _CH)
    pc = 128
    nq = rows_pt // pc

    def body(ind, x2, src2, dst2, zero, s0, u0a, u0b, dis2,
             sidx, didx, rows, pind, px, pdis, pu, acc, gsem, ssem):
        c = lax.axis_index("c")
        s = lax.axis_index("s")
        pltpu.sync_copy(zero, acc.at[pl.ds(s * rows_pt, rows_pt)])
        lane = lax.iota(jnp.int32, 16)

        def pro_all(u0x):
            pltpu.sync_copy(zero.at[pl.ds(0, pc)], pu)

            def chunk(q, carry):
                base = s * rows_pt + q * pc
                pltpu.sync_copy(ind.at[0, pl.ds(base, pc)], pind.at[0])
                pltpu.sync_copy(ind.at[1, pl.ds(base, pc)], pind.at[1])
                pltpu.sync_copy(x2.at[pl.ds(base, pc)], px)

                def vstep(i, carry2):
                    o = i * 16
                    rl = o + lane
                    d = _rsqrt16(pind[0, pl.ds(o, 16)]
                                 + pind[1, pl.ds(o, 16)] + 1.0)
                    pdis[pl.ds(o, 16)] = d
                    z16i = jnp.zeros((16,), jnp.int32)
                    x0 = plsc.load_gather(px, [rl, z16i])
                    x1 = plsc.load_gather(px, [rl, z16i + 1])
                    plsc.store_scatter(pu, [rl, z16i], d * x0)
                    plsc.store_scatter(pu, [rl, z16i + 1], d * x1)
                    return carry2

                lax.fori_loop(0, pc // 16, vstep, 0)
                pltpu.sync_copy(pu, u0x.at[pl.ds(base, pc)])
                pltpu.sync_copy(pdis, dis2.at[c, pl.ds(base, pc)])
                return carry

            lax.fori_loop(0, nq, chunk, 0)

        @pl.when(c == 0)
        def _():
            pro_all(u0a)

        @pl.when(c == 1)
        def _():
            pro_all(u0b)

        plsc.subcore_barrier()
        first = (c * _NS + s) * nch

        @pl.when(c == 0)
        def _():
            _edge_loop(u0a, src2, dst2, zero, sidx, didx, rows, acc,
                       gsem, ssem, first, nch // _NB, True)

        @pl.when(c == 1)
        def _():
            _edge_loop(u0b, src2, dst2, zero, sidx, didx, rows, acc,
                       gsem, ssem, first, nch // _NB, True)

        plsc.subcore_barrier()
        off = s * rows_pt
        pltpu.sync_copy(acc.at[pl.ds(off, rows_pt)],
                        s0.at[c, pl.ds(off, rows_pt)])

    return pl.kernel(
        body,
        out_type=(jax.ShapeDtypeStruct((_NC, n_pad, 16), _F32),
                  jax.ShapeDtypeStruct((n_pad, 16), _F32),
                  jax.ShapeDtypeStruct((n_pad, 16), _F32),
                  jax.ShapeDtypeStruct((_NC, n_pad), _F32)),
        mesh=_MESH,
        compiler_params=_SC_PARAMS,
        scratch_types=[
            pltpu.VMEM((2, _NB, _CH), jnp.int32),
            pltpu.VMEM((2, _NB, _CH), jnp.int32),
            pltpu.VMEM((2, _NB, _CH, 16), _F32),
            pltpu.VMEM((2, pc), _F32),
            pltpu.VMEM((pc, 2), _F32),
            pltpu.VMEM((pc,), _F32),
            pltpu.VMEM((pc, 16), _F32),
            pltpu.VMEM_SHARED((n_pad, 16), _F32),
            pltpu.SemaphoreType.DMA,
            pltpu.SemaphoreType.DMA,
        ],
    )


@functools.lru_cache(maxsize=None)
def _sc_pass1(n_pad, e_pad):
    rows_pt = n_pad // _NS
    nch = e_pad // (_NS * _CH)
    pc = 128
    nq = rows_pt // pc

    def body(s0, u0a, u0b, dis2, w1ah, w1bh, src2, dst2, zero,
             s1, u1a, u1b, p0t,
             sidx, didx, rows, ps0a, ps0b, pu0, pdis, pua, pub, pp0, pw,
             acc, gsem, ssem):
        c = lax.axis_index("c")
        s = lax.axis_index("s")
        pltpu.sync_copy(zero, acc.at[pl.ds(s * rows_pt, rows_pt)])
        pltpu.sync_copy(w1ah, pw.at[0])
        pltpu.sync_copy(w1bh, pw.at[1])

        def pro_all(u0x, u1ax, u1bx):
            w1a0 = pw[0, 0]
            w1a1 = pw[0, 1]
            b1a = pw[0, 2]
            w1b0 = pw[1, 0]
            w1b1 = pw[1, 1]
            b1b = pw[1, 2]

            lane = lax.iota(jnp.int32, 16)

            def chunk(q, carry):
                base = s * rows_pt + q * pc
                pltpu.sync_copy(s0.at[0, pl.ds(base, pc)], ps0a)
                pltpu.sync_copy(s0.at[1, pl.ds(base, pc)], ps0b)
                pltpu.sync_copy(u0x.at[pl.ds(base, pc)], pu0)
                pltpu.sync_copy(dis2.at[c, pl.ds(base, pc)], pdis)

                def gstep(i, carry2):
                    o = i * 16
                    rl = o + lane
                    z16i = jnp.zeros((16,), jnp.int32)
                    dv = pdis[pl.ds(o, 16)]
                    p00v = dv * (plsc.load_gather(ps0a, [rl, z16i])
                                 + plsc.load_gather(ps0b, [rl, z16i])
                                 + plsc.load_gather(pu0, [rl, z16i]))
                    p01v = dv * (plsc.load_gather(ps0a, [rl, z16i + 1])
                                 + plsc.load_gather(ps0b, [rl, z16i + 1])
                                 + plsc.load_gather(pu0, [rl, z16i + 1]))
                    pp0[0, pl.ds(o, 16)] = p00v
                    pp0[1, pl.ds(o, 16)] = p01v
                    for k in range(16):
                        ik = jnp.full((16,), k, jnp.int32)
                        ha = jnp.maximum(p00v * w1a0[ik] + p01v * w1a1[ik]
                                         + b1a[ik], 0.0)
                        plsc.store_scatter(pua, [rl, ik], dv * ha)
                        hb = jnp.maximum(p00v * w1b0[ik] + p01v * w1b1[ik]
                                         + b1b[ik], 0.0)
                        plsc.store_scatter(pub, [rl, ik], dv * hb)
                    return carry2

                lax.fori_loop(0, pc // 16, gstep, 0)
                pltpu.sync_copy(pua, u1ax.at[pl.ds(base, pc)])
                pltpu.sync_copy(pub, u1bx.at[pl.ds(base, pc)])
                pltpu.sync_copy(pp0.at[0], p0t.at[c, 0, pl.ds(base, pc)])
                pltpu.sync_copy(pp0.at[1], p0t.at[c, 1, pl.ds(base, pc)])
                return carry

            lax.fori_loop(0, nq, chunk, 0)

        @pl.when(c == 0)
        def _():
            pro_all(u0a, u1a.at[0], u1b.at[0])

        @pl.when(c == 1)
        def _():
            pro_all(u0b, u1a.at[1], u1b.at[1])

        plsc.subcore_barrier()
        first = s * nch

        @pl.when(c == 0)
        def _():
            _edge_loop(u1a.at[0], src2, dst2, zero, sidx, didx, rows, acc,
                       gsem, ssem, first, nch // _NB, True)

        @pl.when(c == 1)
        def _():
            _edge_loop(u1b.at[1], src2, dst2, zero, sidx, didx, rows, acc,
                       gsem, ssem, first, nch // _NB, True)

        plsc.subcore_barrier()
        off = s * rows_pt
        pltpu.sync_copy(acc.at[pl.ds(off, rows_pt)],
                        s1.at[c, pl.ds(off, rows_pt)])

    return pl.kernel(
        body,
        out_type=(jax.ShapeDtypeStruct((_NC, n_pad, 16), _F32),
                  jax.ShapeDtypeStruct((_NC, n_pad, 16), _F32),
                  jax.ShapeDtypeStruct((_NC, n_pad, 16), _F32),
                  jax.ShapeDtypeStruct((_NC, 2, n_pad), _F32)),
        mesh=_MESH,
        compiler_params=_SC_PARAMS,
        scratch_types=[
            pltpu.VMEM((2, _NB, _CH), jnp.int32),
            pltpu.VMEM((2, _NB, _CH), jnp.int32),
            pltpu.VMEM((2, _NB, _CH, 16), _F32),
            pltpu.VMEM((pc, 16), _F32),
            pltpu.VMEM((pc, 16), _F32),
            pltpu.VMEM((pc, 16), _F32),
            pltpu.VMEM((pc,), _F32),
            pltpu.VMEM((pc, 16), _F32),
            pltpu.VMEM((pc, 16), _F32),
            pltpu.VMEM((2, pc), _F32),
            pltpu.VMEM((2, 3, 16), _F32),
            pltpu.VMEM_SHARED((n_pad, 16), _F32),
            pltpu.SemaphoreType.DMA,
            pltpu.SemaphoreType.DMA,
        ],
    )


@functools.lru_cache(maxsize=None)
def _sc_pass3(n_pad, e_pad):
    rows_pt = n_pad // _NS
    nch = e_pad // (_NS * _CH)
    half = n_pad // _NC
    rows_ph = half // _NS
    pc = 784
    nqe = rows_ph // pc

    def body(u3, dis2, b3v, src2, dst2, zero, out,
             sidx, didx, rows, pacc, pu3, pdis, pb3, acc, gsem, ssem):
        c = lax.axis_index("c")
        s = lax.axis_index("s")
        pltpu.sync_copy(zero, acc.at[pl.ds(s * rows_pt, rows_pt)])
        pltpu.sync_copy(b3v, pb3)
        plsc.subcore_barrier()
        _edge_loop(u3, src2, dst2, zero, sidx, didx, rows, acc, gsem, ssem,
                   s * nch, nch // _NB, True)
        plsc.subcore_barrier()

        bv = pb3[pl.ds(0, 16)]

        def chunk(q, carry):
            base = c * half + s * rows_ph + q * pc
            pltpu.sync_copy(acc.at[pl.ds(base, pc)], pacc)
            pltpu.sync_copy(u3.at[pl.ds(base, pc)], pu3)
            pltpu.sync_copy(dis2.at[c, pl.ds(base, pc)], pdis)

            def vstep(i, carry2):
                o = i * 16
                pacc[pl.ds(o, 16)] = (pdis[pl.ds(o, 16)]
                                      * (pacc[pl.ds(o, 16)]
                                         + pu3[pl.ds(o, 16)]) + bv)
                return carry2

            lax.fori_loop(0, pc // 16, vstep, 0)
            pltpu.sync_copy(pacc, out.at[pl.ds(base, pc)])
            return carry

        lax.fori_loop(0, nqe, chunk, 0)

    return pl.kernel(
        body,
        out_type=jax.ShapeDtypeStruct((n_pad,), _F32),
        mesh=_MESH,
        compiler_params=_SC_PARAMS,
        scratch_types=[
            pltpu.VMEM((2, _NB, _CH), jnp.int32),
            pltpu.VMEM((2, _NB, _CH), jnp.int32),
            pltpu.VMEM((2, _NB, _CH), _F32),
            pltpu.VMEM((pc,), _F32),
            pltpu.VMEM((pc,), _F32),
            pltpu.VMEM((pc,), _F32),
            pltpu.VMEM((16,), _F32),
            pltpu.VMEM_SHARED((n_pad,), _F32),
            pltpu.SemaphoreType.DMA,
            pltpu.SemaphoreType.DMA,
        ],
    )


def _tc_c(s1, u1a0, u1b1, dis, p0t, x, W2, b2, W3):
    def body(s1_ref, ua_ref, ub_ref, dis_ref, p0_ref, x_ref,
             w2_ref, b2_ref, w3_ref, u3_ref):
        d = dis_ref[...]
        aha = d * (s1_ref[0] + ua_ref[...])
        ahb = d * (s1_ref[1] + ub_ref[...])
        q = jnp.concatenate([aha, ahb[:, :14], p0_ref[0], p0_ref[1]], axis=1)
        h2 = jnp.maximum(jnp.dot(q, w2_ref[...],
                                 preferred_element_type=_F32) + b2_ref[...],
                         0.0)
        t3 = (jnp.dot(h2, w3_ref[...][:30], preferred_element_type=_F32)
              + jnp.dot(x_ref[...], w3_ref[...][30:],
                        preferred_element_type=_F32))
        u3_ref[...] = d * t3

    n_pad = x.shape[0]
    row = lambda k: pl.BlockSpec((_BLK,) + k, lambda i: (i,) + (0,) * len(k))
    pair = lambda k: pl.BlockSpec((_NC, _BLK) + k,
                                  lambda i: (0, i) + (0,) * len(k))
    full = lambda sh: pl.BlockSpec(sh, lambda i: (0,) * len(sh))
    return pl.pallas_call(
        body,
        grid=(n_pad // _BLK,),
        in_specs=[pair((16,)), row((16,)), row((16,)), row((1,)),
                  pair((1,)), row((2,)),
                  full((32, 30)), full((1, 30)), full((32, 1))],
        out_specs=[row((1,))],
        out_shape=[jax.ShapeDtypeStruct((n_pad, 1), _F32)],
    )(s1, u1a0, u1b1, dis, p0t, x, W2, b2, W3)[0]


def kernel(x, edge_index, W1, b1, W2, b2, W3, b3):
    n = x.shape[0]
    e = edge_index.shape[1]
    n_pad = _ceil_to(n + 64, _BLK)
    e_pad = _ceil_to(e, _NC * _NS * _CH * _NB)
    spare = n_pad - n

    pad = n + jnp.arange(e_pad - e, dtype=jnp.int32) % spare
    src = jnp.concatenate([edge_index[0], pad]).reshape(e_pad // _CH, _CH)
    dst = jnp.concatenate([edge_index[1], pad]).reshape(e_pad // _CH, _CH)
    x_pad = jnp.zeros((n_pad, 2), _F32).at[:n].set(x)

    rows_pt = n_pad // _NS
    ones_ch = jnp.ones((_CH,), _F32)
    z1 = jnp.zeros((rows_pt,), _F32)
    z16 = jnp.zeros((rows_pt, 16), _F32)

    w1p = jnp.zeros((2, 32), _F32).at[:, :30].set(W1)
    b1p = jnp.zeros((32,), _F32).at[:30].set(b1)
    w1ah = jnp.stack([w1p[0, :16], w1p[1, :16], b1p[:16]])
    w1bh = jnp.stack([w1p[0, 16:], w1p[1, 16:], b1p[16:]])
    b3v = jnp.full((16,), b3[0], _F32)

    ind = _sc_deg(n_pad, e_pad)(ones_ch, src, dst, z1)
    s0, u0a, u0b, dis2 = _sc_pass0(n_pad, e_pad)(ind, x_pad, src, dst, z16)
    s1, u1a, u1b, p0t = _sc_pass1(n_pad, e_pad)(
        s0, u0a, u0b, dis2, w1ah, w1bh, src, dst, z16)
    u3 = _tc_c(s1, u1a[0], u1b[1], dis2[0].reshape(n_pad, 1),
               p0t[0].reshape(_NC, n_pad, 1), x_pad, W2,
               b2.reshape(1, 30), W3)
    out = _sc_pass3(n_pad, e_pad)(u3.reshape(n_pad), dis2, b3v, src, dst, z1)
    return out[:n].reshape(n, 1)


# speedup vs baseline: 1.0649x; 1.0649x over previous
import functools

import numpy as np

import jax
import jax.numpy as jnp
from jax import lax
from jax.experimental import pallas as pl
from jax.experimental.pallas import tpu as pltpu
from jax.experimental.pallas import tpu_sc as plsc

_F32 = jnp.float32
_CH = 128
_NB = 4
_NC, _NS = 2, 16
_BLK = 2048


def _ceil_to(v, m):
    return -(-v // m) * m


def _rsqrt16(z):
    i = plsc.bitcast(z, jnp.int32)
    y = plsc.bitcast(jnp.int32(0x5F3759DF) - (i >> 1), _F32)
    for _ in range(3):
        y = y * (1.5 - 0.5 * z * y * y)
    return y


_MESH = plsc.VectorSubcoreMesh(core_axis_name="c", subcore_axis_name="s")
_SC_PARAMS = pltpu.CompilerParams(use_tc_tiling_on_sc=False,
                                  needs_layout_passes=False)


def _drain_slot(zero, rows, ssem, p):
    for j in range(_NB):
        pltpu.make_async_copy(zero.at[pl.ds(0, _CH)], rows.at[p, j],
                              ssem).wait()


def _edge_loop(uu, src2, dst2, zero, sidx, didx, rows, acc, gsem, ssem,
               first, nb, gather):
    def it(b, carry):
        p = jnp.remainder(b, 2)

        @pl.when(b >= 2)
        def _():
            _drain_slot(zero, rows, ssem, p)

        row0 = first + b * _NB
        pltpu.sync_copy(dst2.at[pl.ds(row0, _NB)], didx.at[p])
        if gather:
            pltpu.sync_copy(src2.at[pl.ds(row0, _NB)], sidx.at[p])
            ds = [pltpu.async_copy(uu.at[sidx.at[p, j]], rows.at[p, j], gsem)
                  for j in range(_NB)]
            for d in ds:
                d.wait()
        for j in range(_NB):
            pltpu.async_copy(rows.at[p, j], acc.at[didx.at[p, j]],
                             ssem, add=True)
        return carry

    lax.fori_loop(0, nb, it, 0)
    for p in range(2):
        _drain_slot(zero, rows, ssem, p)


@functools.lru_cache(maxsize=None)
def _sc_deg(n_pad, e_pad):
    rows_pt = n_pad // _NS
    nch = e_pad // (_NC * _NS * _CH)

    def body(ones, src2, dst2, zero, out, sidx, didx, rows, acc, gsem, ssem):
        c = lax.axis_index("c")
        s = lax.axis_index("s")
        pltpu.sync_copy(zero, acc.at[pl.ds(s * rows_pt, rows_pt)])
        for p in range(2):
            for j in range(_NB):
                pltpu.sync_copy(ones, rows.at[p, j])
        plsc.subcore_barrier()
        _edge_loop(None, src2, dst2, zero, sidx, didx, rows, acc, gsem, ssem,
                   (c * _NS + s) * nch, nch // _NB, False)
        plsc.subcore_barrier()
        off = s * rows_pt
        pltpu.sync_copy(acc.at[pl.ds(off, rows_pt)],
                        out.at[c, pl.ds(off, rows_pt)])

    return pl.kernel(
        body,
        out_type=jax.ShapeDtypeStruct((_NC, n_pad), _F32),
        mesh=_MESH,
        compiler_params=_SC_PARAMS,
        scratch_types=[
            pltpu.VMEM((2, _NB, _CH), jnp.int32),
            pltpu.VMEM((2, _NB, _CH), jnp.int32),
            pltpu.VMEM((2, _NB, _CH), _F32),
            pltpu.VMEM_SHARED((n_pad,), _F32),
            pltpu.SemaphoreType.DMA,
            pltpu.SemaphoreType.DMA,
        ],
    )


@functools.lru_cache(maxsize=None)
def _sc_pass0(n_pad, e_pad):
    rows_pt = n_pad // _NS
    nch = e_pad // (_NC * _NS * _CH)
    pc = 128
    nq = rows_pt // pc

    def body(ind, x2, src2, dst2, zero, s0, u0a, u0b, dis2,
             sidx, didx, rows, pind, px, pdis, pu, acc, gsem, ssem):
        c = lax.axis_index("c")
        s = lax.axis_index("s")
        pltpu.sync_copy(zero, acc.at[pl.ds(s * rows_pt, rows_pt)])
        lane = lax.iota(jnp.int32, 16)

        def pro_all(u0x):
            pltpu.sync_copy(zero.at[pl.ds(0, pc)], pu)

            def chunk(q, carry):
                base = s * rows_pt + q * pc
                pltpu.sync_copy(ind.at[0, pl.ds(base, pc)], pind.at[0])
                pltpu.sync_copy(ind.at[1, pl.ds(base, pc)], pind.at[1])
                pltpu.sync_copy(x2.at[pl.ds(base, pc)], px)

                def vstep(i, carry2):
                    o = i * 16
                    rl = o + lane
                    d = _rsqrt16(pind[0, pl.ds(o, 16)]
                                 + pind[1, pl.ds(o, 16)] + 1.0)
                    pdis[pl.ds(o, 16)] = d
                    z16i = jnp.zeros((16,), jnp.int32)
                    x0 = plsc.load_gather(px, [rl, z16i])
                    x1 = plsc.load_gather(px, [rl, z16i + 1])
                    plsc.store_scatter(pu, [rl, z16i], d * x0)
                    plsc.store_scatter(pu, [rl, z16i + 1], d * x1)
                    return carry2

                lax.fori_loop(0, pc // 16, vstep, 0)
                pltpu.sync_copy(pu, u0x.at[pl.ds(base, pc)])
                pltpu.sync_copy(pdis, dis2.at[c, pl.ds(base, pc)])
                return carry

            lax.fori_loop(0, nq, chunk, 0)

        @pl.when(c == 0)
        def _():
            pro_all(u0a)

        @pl.when(c == 1)
        def _():
            pro_all(u0b)

        plsc.subcore_barrier()
        first = (c * _NS + s) * nch

        @pl.when(c == 0)
        def _():
            _edge_loop(u0a, src2, dst2, zero, sidx, didx, rows, acc,
                       gsem, ssem, first, nch // _NB, True)

        @pl.when(c == 1)
        def _():
            _edge_loop(u0b, src2, dst2, zero, sidx, didx, rows, acc,
                       gsem, ssem, first, nch // _NB, True)

        plsc.subcore_barrier()
        off = s * rows_pt
        pltpu.sync_copy(acc.at[pl.ds(off, rows_pt)],
                        s0.at[c, pl.ds(off, rows_pt)])

    return pl.kernel(
        body,
        out_type=(jax.ShapeDtypeStruct((_NC, n_pad, 16), _F32),
                  jax.ShapeDtypeStruct((n_pad, 16), _F32),
                  jax.ShapeDtypeStruct((n_pad, 16), _F32),
                  jax.ShapeDtypeStruct((_NC, n_pad), _F32)),
        mesh=_MESH,
        compiler_params=_SC_PARAMS,
        scratch_types=[
            pltpu.VMEM((2, _NB, _CH), jnp.int32),
            pltpu.VMEM((2, _NB, _CH), jnp.int32),
            pltpu.VMEM((2, _NB, _CH, 16), _F32),
            pltpu.VMEM((2, pc), _F32),
            pltpu.VMEM((pc, 2), _F32),
            pltpu.VMEM((pc,), _F32),
            pltpu.VMEM((pc, 16), _F32),
            pltpu.VMEM_SHARED((n_pad, 16), _F32),
            pltpu.SemaphoreType.DMA,
            pltpu.SemaphoreType.DMA,
        ],
    )


@functools.lru_cache(maxsize=None)
def _sc_pass1(n_pad, e_pad):
    rows_pt = n_pad // _NS
    nch = e_pad // (_NS * _CH)
    pc = 128
    nq = rows_pt // pc

    def body(s0, u0a, u0b, dis2, w1ah, w1bh, src2, dst2, zero,
             s1, u1a, u1b, p0t,
             sidx, didx, rows, ps0a, ps0b, pu0, pdis, pua, pub, pp0, pw,
             acc, gsem, ssem):
        c = lax.axis_index("c")
        s = lax.axis_index("s")
        pltpu.sync_copy(zero, acc.at[pl.ds(s * rows_pt, rows_pt)])
        pltpu.sync_copy(w1ah, pw.at[0])
        pltpu.sync_copy(w1bh, pw.at[1])

        def pro_all(u0x, u1ax, u1bx):
            w1a0 = pw[0, 0]
            w1a1 = pw[0, 1]
            b1a = pw[0, 2]
            w1b0 = pw[1, 0]
            w1b1 = pw[1, 1]
            b1b = pw[1, 2]

            lane = lax.iota(jnp.int32, 16)

            def chunk(q, carry):
                base = s * rows_pt + q * pc
                pltpu.sync_copy(s0.at[0, pl.ds(base, pc)], ps0a)
                pltpu.sync_copy(s0.at[1, pl.ds(base, pc)], ps0b)
                pltpu.sync_copy(u0x.at[pl.ds(base, pc)], pu0)
                pltpu.sync_copy(dis2.at[c, pl.ds(base, pc)], pdis)

                def gstep(i, carry2):
                    o = i * 16
                    rl = o + lane
                    z16i = jnp.zeros((16,), jnp.int32)
                    dv = pdis[pl.ds(o, 16)]
                    p00v = dv * (plsc.load_gather(ps0a, [rl, z16i])
                                 + plsc.load_gather(ps0b, [rl, z16i])
                                 + plsc.load_gather(pu0, [rl, z16i]))
                    p01v = dv * (plsc.load_gather(ps0a, [rl, z16i + 1])
                                 + plsc.load_gather(ps0b, [rl, z16i + 1])
                                 + plsc.load_gather(pu0, [rl, z16i + 1]))
                    pp0[0, pl.ds(o, 16)] = p00v
                    pp0[1, pl.ds(o, 16)] = p01v
                    for k in range(16):
                        ik = jnp.full((16,), k, jnp.int32)
                        ha = jnp.maximum(p00v * w1a0[ik] + p01v * w1a1[ik]
                                         + b1a[ik], 0.0)
                        plsc.store_scatter(pua, [rl, ik], dv * ha)
                        hb = jnp.maximum(p00v * w1b0[ik] + p01v * w1b1[ik]
                                         + b1b[ik], 0.0)
                        plsc.store_scatter(pub, [rl, ik], dv * hb)
                    return carry2

                lax.fori_loop(0, pc // 16, gstep, 0)
                pltpu.sync_copy(pua, u1ax.at[pl.ds(base, pc)])
                pltpu.sync_copy(pub, u1bx.at[pl.ds(base, pc)])
                pltpu.sync_copy(pp0.at[0], p0t.at[c, 0, pl.ds(base, pc)])
                pltpu.sync_copy(pp0.at[1], p0t.at[c, 1, pl.ds(base, pc)])
                return carry

            lax.fori_loop(0, nq, chunk, 0)

        @pl.when(c == 0)
        def _():
            pro_all(u0a, u1a.at[0], u1b.at[0])

        @pl.when(c == 1)
        def _():
            pro_all(u0b, u1a.at[1], u1b.at[1])

        plsc.subcore_barrier()
        first = s * nch

        @pl.when(c == 0)
        def _():
            _edge_loop(u1a.at[0], src2, dst2, zero, sidx, didx, rows, acc,
                       gsem, ssem, first, nch // _NB, True)

        @pl.when(c == 1)
        def _():
            _edge_loop(u1b.at[1], src2, dst2, zero, sidx, didx, rows, acc,
                       gsem, ssem, first, nch // _NB, True)

        plsc.subcore_barrier()

        def ep_chunk(q, carry):
            base = s * rows_pt + q * pc
            pltpu.sync_copy(acc.at[pl.ds(base, pc)], ps0a)

            @pl.when(c == 0)
            def _():
                pltpu.sync_copy(u1a.at[0, pl.ds(base, pc)], ps0b)

            @pl.when(c == 1)
            def _():
                pltpu.sync_copy(u1b.at[1, pl.ds(base, pc)], ps0b)

            def vrow(r, carry2):
                ps0a[r] = ps0a[r] + ps0b[r]
                return carry2

            lax.fori_loop(0, pc, vrow, 0)
            pltpu.sync_copy(ps0a, s1.at[c, pl.ds(base, pc)])
            return carry

        lax.fori_loop(0, nq, ep_chunk, 0)

    return pl.kernel(
        body,
        out_type=(jax.ShapeDtypeStruct((_NC, n_pad, 16), _F32),
                  jax.ShapeDtypeStruct((_NC, n_pad, 16), _F32),
                  jax.ShapeDtypeStruct((_NC, n_pad, 16), _F32),
                  jax.ShapeDtypeStruct((_NC, 2, n_pad), _F32)),
        mesh=_MESH,
        compiler_params=_SC_PARAMS,
        scratch_types=[
            pltpu.VMEM((2, _NB, _CH), jnp.int32),
            pltpu.VMEM((2, _NB, _CH), jnp.int32),
            pltpu.VMEM((2, _NB, _CH, 16), _F32),
            pltpu.VMEM((pc, 16), _F32),
            pltpu.VMEM((pc, 16), _F32),
            pltpu.VMEM((pc, 16), _F32),
            pltpu.VMEM((pc,), _F32),
            pltpu.VMEM((pc, 16), _F32),
            pltpu.VMEM((pc, 16), _F32),
            pltpu.VMEM((2, pc), _F32),
            pltpu.VMEM((2, 3, 16), _F32),
            pltpu.VMEM_SHARED((n_pad, 16), _F32),
            pltpu.SemaphoreType.DMA,
            pltpu.SemaphoreType.DMA,
        ],
    )


@functools.lru_cache(maxsize=None)
def _sc_pass3(n_pad, e_pad):
    rows_pt = n_pad // _NS
    nch = e_pad // (_NS * _CH)
    half = n_pad // _NC
    rows_ph = half // _NS
    pc = 784
    nqe = rows_ph // pc

    def body(u3, dis2, b3v, src2, dst2, zero, out,
             sidx, didx, rows, pacc, pu3, pdis, pb3, acc, gsem, ssem):
        c = lax.axis_index("c")
        s = lax.axis_index("s")
        pltpu.sync_copy(zero, acc.at[pl.ds(s * rows_pt, rows_pt)])
        pltpu.sync_copy(b3v, pb3)
        plsc.subcore_barrier()
        _edge_loop(u3, src2, dst2, zero, sidx, didx, rows, acc, gsem, ssem,
                   s * nch, nch // _NB, True)
        plsc.subcore_barrier()

        bv = pb3[pl.ds(0, 16)]

        def chunk(q, carry):
            base = c * half + s * rows_ph + q * pc
            pltpu.sync_copy(acc.at[pl.ds(base, pc)], pacc)
            pltpu.sync_copy(u3.at[pl.ds(base, pc)], pu3)
            pltpu.sync_copy(dis2.at[c, pl.ds(base, pc)], pdis)

            def vstep(i, carry2):
                o = i * 16
                pacc[pl.ds(o, 16)] = (pdis[pl.ds(o, 16)]
                                      * (pacc[pl.ds(o, 16)]
                                         + pu3[pl.ds(o, 16)]) + bv)
                return carry2

            lax.fori_loop(0, pc // 16, vstep, 0)
            pltpu.sync_copy(pacc, out.at[pl.ds(base, pc)])
            return carry

        lax.fori_loop(0, nqe, chunk, 0)

    return pl.kernel(
        body,
        out_type=jax.ShapeDtypeStruct((n_pad,), _F32),
        mesh=_MESH,
        compiler_params=_SC_PARAMS,
        scratch_types=[
            pltpu.VMEM((2, _NB, _CH), jnp.int32),
            pltpu.VMEM((2, _NB, _CH), jnp.int32),
            pltpu.VMEM((2, _NB, _CH), _F32),
            pltpu.VMEM((pc,), _F32),
            pltpu.VMEM((pc,), _F32),
            pltpu.VMEM((pc,), _F32),
            pltpu.VMEM((16,), _F32),
            pltpu.VMEM_SHARED((n_pad,), _F32),
            pltpu.SemaphoreType.DMA,
            pltpu.SemaphoreType.DMA,
        ],
    )


def _tc_c(s1, dis, p0t, x, W2, b2, W3):
    def body(s1_ref, dis_ref, p0_ref, x_ref,
             w2_ref, b2_ref, w3_ref, u3_ref):
        d = dis_ref[...]
        aha = d * s1_ref[0]
        ahb = d * s1_ref[1]
        q = jnp.concatenate([aha, ahb[:, :14], p0_ref[0], p0_ref[1]], axis=1)
        h2 = jnp.maximum(jnp.dot(q, w2_ref[...],
                                 preferred_element_type=_F32) + b2_ref[...],
                         0.0)
        t3 = (jnp.dot(h2, w3_ref[...][:30], preferred_element_type=_F32)
              + jnp.dot(x_ref[...], w3_ref[...][30:],
                        preferred_element_type=_F32))
        u3_ref[...] = d * t3

    n_pad = x.shape[0]
    row = lambda k: pl.BlockSpec((_BLK,) + k, lambda i: (i,) + (0,) * len(k))
    pair = lambda k: pl.BlockSpec((_NC, _BLK) + k,
                                  lambda i: (0, i) + (0,) * len(k))
    full = lambda sh: pl.BlockSpec(sh, lambda i: (0,) * len(sh))
    return pl.pallas_call(
        body,
        grid=(n_pad // _BLK,),
        in_specs=[pair((16,)), row((1,)),
                  pair((1,)), row((2,)),
                  full((32, 30)), full((1, 30)), full((32, 1))],
        out_specs=[row((1,))],
        out_shape=[jax.ShapeDtypeStruct((n_pad, 1), _F32)],
    )(s1, dis, p0t, x, W2, b2, W3)[0]


def kernel(x, edge_index, W1, b1, W2, b2, W3, b3):
    n = x.shape[0]
    e = edge_index.shape[1]
    n_pad = _ceil_to(n + 64, _BLK)
    e_pad = _ceil_to(e, _NC * _NS * _CH * _NB)
    spare = n_pad - n

    pad = jnp.asarray(n + np.arange(e_pad - e, dtype=np.int32) % spare)
    src = jnp.concatenate([edge_index[0], pad]).reshape(e_pad // _CH, _CH)
    dst = jnp.concatenate([edge_index[1], pad]).reshape(e_pad // _CH, _CH)
    x_pad = jnp.zeros((n_pad, 2), _F32).at[:n].set(x)

    rows_pt = n_pad // _NS
    ones_ch = jnp.ones((_CH,), _F32)
    z1 = jnp.zeros((rows_pt,), _F32)
    z16 = jnp.zeros((rows_pt, 16), _F32)

    w1p = jnp.zeros((2, 32), _F32).at[:, :30].set(W1)
    b1p = jnp.zeros((32,), _F32).at[:30].set(b1)
    w1ah = jnp.stack([w1p[0, :16], w1p[1, :16], b1p[:16]])
    w1bh = jnp.stack([w1p[0, 16:], w1p[1, 16:], b1p[16:]])
    b3v = jnp.full((16,), b3[0], _F32)

    ind = _sc_deg(n_pad, e_pad)(ones_ch, src, dst, z1)
    s0, u0a, u0b, dis2 = _sc_pass0(n_pad, e_pad)(ind, x_pad, src, dst, z16)
    s1, u1a, u1b, p0t = _sc_pass1(n_pad, e_pad)(
        s0, u0a, u0b, dis2, w1ah, w1bh, src, dst, z16)
    u3 = _tc_c(s1, dis2[0].reshape(n_pad, 1),
               p0t[0].reshape(_NC, n_pad, 1), x_pad, W2,
               b2.reshape(1, 30), W3)
    out = _sc_pass3(n_pad, e_pad)(u3.reshape(n_pad), dis2, b3v, src, dst, z1)
    return out[:n].reshape(n, 1)
